# baseline (speedup 1.0000x reference)
import functools

import jax
import jax.numpy as jnp
from jax import lax
from jax.experimental import pallas as pl
from jax.experimental.pallas import tpu as pltpu
from jax.experimental.pallas import tpu_sc as plsc

ALPHA = 0.5
NEG = -3.0e38
LANES = 16
NW = 32


def _leaky(x):
    return jnp.where(x > 0, x, 0.2 * x)


def _prep_kernel(intent_ref, wk_ref, k_ref):
    k_ref[...] = lax.dot_general(intent_ref[...], wk_ref[...],
                                 (((1,), (1,)), ((), ())),
                                 preferred_element_type=jnp.float32)


def _pass1_kernel(item_ref, wq_ref, k_ref, intent_ref, wa_ref,
                  idx_ref, ev_ref, ss_ref):
    i = pl.program_id(0)
    blk = item_ref[...]
    B, d = blk.shape
    n_int = k_ref.shape[0]

    q = lax.dot_general(blk, wq_ref[...], (((1,), (1,)), ((), ())),
                        preferred_element_type=jnp.float32)
    logits = lax.dot_general(q, k_ref[...], (((1,), (1,)), ((), ())),
                             preferred_element_type=jnp.float32)
    iota = lax.broadcasted_iota(jnp.int32, (B, n_int), 1)

    def amax(l):
        m = jnp.max(l, axis=1, keepdims=True)
        return jnp.min(jnp.where(l == m, iota, n_int + 1), axis=1)

    i1 = amax(logits)
    l2 = jnp.where(iota == i1[:, None], NEG, logits)
    i2 = amax(l2)
    l3 = jnp.where(iota == i2[:, None], NEG, l2)
    i3 = amax(l3)

    smin = jnp.minimum(jnp.minimum(i1, i2), i3)
    smax = jnp.maximum(jnp.maximum(i1, i2), i3)
    smid = i1 + i2 + i3 - smin - smax

    oh1 = iota == smin[:, None]
    oh2 = iota == smid[:, None]
    oh3 = iota == smax[:, None]

    idx_ref[0, 0, :] = smin
    idx_ref[0, 1, :] = smid
    idx_ref[0, 2, :] = smax

    a1 = lax.dot_general(blk * wa_ref[...][None, :], intent_ref[...],
                         (((1,), (1,)), ((), ())),
                         preferred_element_type=jnp.float32)
    e1 = jnp.exp(_leaky(jnp.sum(jnp.where(oh1, a1, 0.0), axis=1)))
    e2 = jnp.exp(_leaky(jnp.sum(jnp.where(oh2, a1, 0.0), axis=1)))
    e3 = jnp.exp(_leaky(jnp.sum(jnp.where(oh3, a1, 0.0), axis=1)))

    ev_ref[0, 0, :] = e1
    ev_ref[0, 1, :] = e2
    ev_ref[0, 2, :] = e3

    @pl.when(i == 0)
    def _init():
        ss_ref[...] = jnp.zeros_like(ss_ref)

    ss_ref[...] += (jnp.sum(jnp.where(oh1, e1[:, None], 0.0), axis=0)
                    + jnp.sum(jnp.where(oh2, e2[:, None], 0.0), axis=0)
                    + jnp.sum(jnp.where(oh3, e3[:, None], 0.0), axis=0))


CR = 128
NRG = 8


def _make_sc_scatter(n, d, n_int):
    NCG = d // 128
    dc = 128
    nreg = dc // LANES
    rpg = (n + NRG - 1) // NRG
    rows_rg = (rpg + 4 * CR - 1) // (4 * CR) * (4 * CR)
    half = rows_rg // 2
    nch = half // CR
    mesh = plsc.VectorSubcoreMesh(core_axis_name="c", subcore_axis_name="s")

    @functools.partial(
        pl.kernel, mesh=mesh,
        compiler_params=pltpu.CompilerParams(needs_layout_passes=False),
        out_type=jax.ShapeDtypeStruct((NRG, NCG, n_int * dc), jnp.float32),
        scratch_types=[
            pltpu.VMEM((half,), jnp.int32),
            pltpu.VMEM((half,), jnp.int32),
            pltpu.VMEM((half,), jnp.int32),
            pltpu.VMEM((half,), jnp.float32),
            pltpu.VMEM((half,), jnp.float32),
            pltpu.VMEM((half,), jnp.float32),
            pltpu.VMEM((CR, dc), jnp.float32),
            pltpu.VMEM((CR, dc), jnp.float32),
            pltpu.VMEM((n_int * dc,), jnp.float32),
            pltpu.SemaphoreType.DMA,
            pltpu.SemaphoreType.DMA,
        ],
    )
    def sc_scatter(i1h, i2h, i3h, e1h, e2h, e3h, items, out,
                   i1v, i2v, i3v, e1v, e2v, e3v, ib0, ib1, acc, sm0, sm1):
        c = lax.axis_index("c")
        s = lax.axis_index("s")
        w = s * 2 + c
        rg = w // NCG
        cg = w % NCG
        rbase = rg * rows_rg

        zero = jnp.zeros((LANES,), jnp.float32)

        def zero_body(t, carry):
            for r in range(nreg):
                acc[pl.ds(t * dc + r * LANES, LANES)] = zero
            return carry

        lax.fori_loop(0, n_int, zero_body, 0)

        dn = lax.GatherDimensionNumbers(
            offset_dims=(), collapsed_slice_dims=(0,),
            start_index_map=(0,))
        pib = lax.GatherScatterMode.PROMISE_IN_BOUNDS
        lane = lax.iota(jnp.int32, LANES)

        def fire(cstart, buf, sem):
            roff = jnp.minimum(cstart, n - CR)
            pltpu.async_copy(
                items.at[pl.ds(roff, CR), pl.ds(cg * dc, dc)], buf, sem)

        def wait(buf, sem):
            pltpu.make_async_copy(
                items.at[pl.ds(0, CR), pl.ds(cg * dc, dc)], buf, sem).wait()

        def compute(cstart, sb, buf):
            roff = jnp.minimum(cstart, n - CR)
            q = roff - sb

            def group_body(g, carry2):
                iv1 = i1v[pl.ds(q + g * LANES, LANES)]
                iv2 = i2v[pl.ds(q + g * LANES, LANES)]
                iv3 = i3v[pl.ds(q + g * LANES, LANES)]
                ev1 = e1v[pl.ds(q + g * LANES, LANES)]
                ev2 = e2v[pl.ds(q + g * LANES, LANES)]
                ev3 = e3v[pl.ds(q + g * LANES, LANES)]

                def item_body(j2, carry3):
                    for u in range(2):
                        j = j2 * 2 + u
                        p = jnp.full((LANES, 1), j, jnp.int32)
                        ok = (roff + g * LANES + j) >= cstart
                        t1 = lax.gather(iv1, p, dn, (1,), mode=pib)
                        t2 = lax.gather(iv2, p, dn, (1,), mode=pib)
                        t3 = lax.gather(iv3, p, dn, (1,), mode=pib)
                        g1 = jnp.where(ok, lax.gather(ev1, p, dn, (1,), mode=pib), 0.0)
                        g2 = jnp.where(ok, lax.gather(ev2, p, dn, (1,), mode=pib), 0.0)
                        g3 = jnp.where(ok, lax.gather(ev3, p, dn, (1,), mode=pib), 0.0)
                        a1 = t1 * dc + lane
                        a2 = t2 * dc + lane
                        a3 = t3 * dc + lane
                        jj = g * LANES + j
                        for r in range(nreg):
                            off = r * LANES
                            rv = buf[jj, pl.ds(off, LANES)]
                            plsc.addupdate_scatter(acc, [a1 + off], rv * g1)
                            plsc.addupdate_scatter(acc, [a2 + off], rv * g2)
                            plsc.addupdate_scatter(acc, [a3 + off], rv * g3)
                    return carry3

                return lax.fori_loop(0, LANES // 2, item_body, carry2)

            lax.fori_loop(0, CR // LANES, group_body, 0)

        for h in range(2):
            hbase = rbase + h * half
            sb = jnp.minimum(hbase, n - half)
            pltpu.sync_copy(i1h.at[pl.ds(sb, half)], i1v)
            pltpu.sync_copy(i2h.at[pl.ds(sb, half)], i2v)
            pltpu.sync_copy(i3h.at[pl.ds(sb, half)], i3v)
            pltpu.sync_copy(e1h.at[pl.ds(sb, half)], e1v)
            pltpu.sync_copy(e2h.at[pl.ds(sb, half)], e2v)
            pltpu.sync_copy(e3h.at[pl.ds(sb, half)], e3v)

            fire(hbase, ib0, sm0)

            def pair_body(m, carry, hbase=hbase, sb=sb):
                c0 = hbase + (2 * m) * CR
                wait(ib0, sm0)
                fire(c0 + CR, ib1, sm1)
                compute(c0, sb, ib0)
                wait(ib1, sm1)

                @pl.when(m + 1 < nch // 2)
                def _prefetch():
                    fire(c0 + 2 * CR, ib0, sm0)

                compute(c0 + CR, sb, ib1)
                return carry

            lax.fori_loop(0, nch // 2, pair_body, 0)

        pltpu.sync_copy(acc, out.at[rg, cg])

    return sc_scatter


def _pass2_kernel(item_ref, idx_ref, acc_ref, ss_ref, wb_ref, out_ref):
    blk = item_ref[...]
    B, d = blk.shape
    n_int = acc_ref.shape[2]

    red = jnp.sum(acc_ref[...], axis=0)
    ncg = red.shape[0]
    full = jnp.concatenate([red[i] for i in range(ncg)], axis=1)
    ss = ss_ref[...]
    denom = jnp.where(ss == 0.0, 1.0, ss)
    intent_new = full / denom[:, None]

    iota = lax.broadcasted_iota(jnp.int32, (B, n_int), 1)
    oh1 = iota == idx_ref[0, 0, :][:, None]
    oh2 = iota == idx_ref[0, 1, :][:, None]
    oh3 = iota == idx_ref[0, 2, :][:, None]

    a2 = lax.dot_general(blk * wb_ref[...][None, :], intent_new,
                         (((1,), (1,)), ((), ())),
                         preferred_element_type=jnp.float32)
    t1 = _leaky(jnp.sum(jnp.where(oh1, a2, 0.0), axis=1))
    t2 = _leaky(jnp.sum(jnp.where(oh2, a2, 0.0), axis=1))
    t3 = _leaky(jnp.sum(jnp.where(oh3, a2, 0.0), axis=1))
    f1 = jnp.exp(t1)
    f2 = jnp.exp(t2)
    f3 = jnp.exp(t3)
    srow = f1 + f2 + f3
    w1 = f1 / srow
    w2 = f2 / srow
    w3 = f3 / srow

    P2 = (w1[:, None] * oh1.astype(jnp.float32)
          + w2[:, None] * oh2.astype(jnp.float32)
          + w3[:, None] * oh3.astype(jnp.float32))
    nei = lax.dot_general(P2, intent_new, (((1,), (0,)), ((), ())),
                          preferred_element_type=jnp.float32)
    out_ref[...] = ALPHA * blk + (1.0 - ALPHA) * nei


def _pick_block(n):
    for b in range(2048, 0, -8):
        if n % b == 0:
            return b
    return n


def kernel(item_emb, n_items, intent_emb, n_intents, Wq, Wk, wa, wb):
    n, d = item_emb.shape
    n_int = intent_emb.shape[0]
    B = _pick_block(n)
    nb = n // B

    k_mat = pl.pallas_call(
        _prep_kernel,
        out_shape=jax.ShapeDtypeStruct((n_int, d), jnp.float32),
    )(intent_emb, Wk)

    grid = (nb,)
    idx, ev, ss = pl.pallas_call(
        _pass1_kernel,
        grid=grid,
        in_specs=[
            pl.BlockSpec((B, d), lambda i: (i, 0)),
            pl.BlockSpec((d, d), lambda i: (0, 0)),
            pl.BlockSpec((n_int, d), lambda i: (0, 0)),
            pl.BlockSpec((n_int, d), lambda i: (0, 0)),
            pl.BlockSpec((d,), lambda i: (0,)),
        ],
        out_specs=[
            pl.BlockSpec((1, 3, B), lambda i: (i, 0, 0)),
            pl.BlockSpec((1, 3, B), lambda i: (i, 0, 0)),
            pl.BlockSpec((n_int,), lambda i: (0,)),
        ],
        out_shape=[
            jax.ShapeDtypeStruct((nb, 3, B), jnp.int32),
            jax.ShapeDtypeStruct((nb, 3, B), jnp.float32),
            jax.ShapeDtypeStruct((n_int,), jnp.float32),
        ],
        compiler_params=pltpu.CompilerParams(
            dimension_semantics=("arbitrary",)),
    )(item_emb, Wq, k_mat, intent_emb, wa)

    idxf = jnp.transpose(idx, (1, 0, 2)).reshape(3, n)
    evf = jnp.transpose(ev, (1, 0, 2)).reshape(3, n)

    accs = _make_sc_scatter(n, d, n_int)(
        idxf[0], idxf[1], idxf[2], evf[0], evf[1], evf[2], item_emb)
    accs = accs.reshape(NRG, d // 128, n_int, 128)

    out = pl.pallas_call(
        _pass2_kernel,
        grid=grid,
        in_specs=[
            pl.BlockSpec((B, d), lambda i: (i, 0)),
            pl.BlockSpec((1, 3, B), lambda i: (i, 0, 0)),
            pl.BlockSpec((NRG, d // 128, n_int, 128),
                         lambda i: (0, 0, 0, 0)),
            pl.BlockSpec((n_int,), lambda i: (0,)),
            pl.BlockSpec((d,), lambda i: (0,)),
        ],
        out_specs=pl.BlockSpec((B, d), lambda i: (i, 0)),
        out_shape=jax.ShapeDtypeStruct((n, d), jnp.float32),
        compiler_params=pltpu.CompilerParams(
            dimension_semantics=("arbitrary",)),
    )(item_emb, idx, accs, ss, wb)
    return out


# speedup vs baseline: 1.1732x; 1.1732x over previous
import functools

import jax
import jax.numpy as jnp
from jax import lax
from jax.experimental import pallas as pl
from jax.experimental.pallas import tpu as pltpu
from jax.experimental.pallas import tpu_sc as plsc

ALPHA = 0.5
NEG = -3.0e38
LANES = 16
NW = 32


def _leaky(x):
    return jnp.where(x > 0, x, 0.2 * x)


def _prep_kernel(intent_ref, wk_ref, k_ref):
    k_ref[...] = lax.dot_general(intent_ref[...], wk_ref[...],
                                 (((1,), (1,)), ((), ())),
                                 preferred_element_type=jnp.float32)


def _pass1_kernel(item_ref, wq_ref, k_ref, intent_ref, wa_ref,
                  idx_ref, ev_ref, ss_ref):
    i = pl.program_id(0)
    blk = item_ref[...]
    B, d = blk.shape
    n_int = k_ref.shape[0]

    q = lax.dot_general(blk, wq_ref[...], (((1,), (1,)), ((), ())),
                        preferred_element_type=jnp.float32)
    logits = lax.dot_general(q, k_ref[...], (((1,), (1,)), ((), ())),
                             preferred_element_type=jnp.float32)
    iota = lax.broadcasted_iota(jnp.int32, (B, n_int), 1)

    def amax(l):
        m = jnp.max(l, axis=1, keepdims=True)
        return jnp.min(jnp.where(l == m, iota, n_int + 1), axis=1)

    i1 = amax(logits)
    l2 = jnp.where(iota == i1[:, None], NEG, logits)
    i2 = amax(l2)
    l3 = jnp.where(iota == i2[:, None], NEG, l2)
    i3 = amax(l3)

    smin = jnp.minimum(jnp.minimum(i1, i2), i3)
    smax = jnp.maximum(jnp.maximum(i1, i2), i3)
    smid = i1 + i2 + i3 - smin - smax

    oh1 = iota == smin[:, None]
    oh2 = iota == smid[:, None]
    oh3 = iota == smax[:, None]

    idx_ref[0, 0, :] = smin
    idx_ref[0, 1, :] = smid
    idx_ref[0, 2, :] = smax

    a1 = lax.dot_general(blk * wa_ref[...][None, :], intent_ref[...],
                         (((1,), (1,)), ((), ())),
                         preferred_element_type=jnp.float32)
    e1 = jnp.exp(_leaky(jnp.sum(jnp.where(oh1, a1, 0.0), axis=1)))
    e2 = jnp.exp(_leaky(jnp.sum(jnp.where(oh2, a1, 0.0), axis=1)))
    e3 = jnp.exp(_leaky(jnp.sum(jnp.where(oh3, a1, 0.0), axis=1)))

    ev_ref[0, 0, :] = e1
    ev_ref[0, 1, :] = e2
    ev_ref[0, 2, :] = e3

    @pl.when(i == 0)
    def _init():
        ss_ref[...] = jnp.zeros_like(ss_ref)

    ss_ref[...] += (jnp.sum(jnp.where(oh1, e1[:, None], 0.0), axis=0)
                    + jnp.sum(jnp.where(oh2, e2[:, None], 0.0), axis=0)
                    + jnp.sum(jnp.where(oh3, e3[:, None], 0.0), axis=0))


CR = 128
NRG = 8


def _make_sc_scatter(n, d, n_int, nbase):
    NCG = d // 128
    dc = 128
    nreg = dc // LANES
    rpg = (n + NRG - 1) // NRG
    rows_rg = (rpg + 4 * CR - 1) // (4 * CR) * (4 * CR)
    half = rows_rg // 2
    nch = half // CR
    mesh = plsc.VectorSubcoreMesh(core_axis_name="c", subcore_axis_name="s")

    @functools.partial(
        pl.kernel, mesh=mesh,
        compiler_params=pltpu.CompilerParams(needs_layout_passes=False),
        out_type=jax.ShapeDtypeStruct((NRG, NCG, n_int, dc), jnp.float32),
        scratch_types=[
            pltpu.VMEM((half,), jnp.int32),
            pltpu.VMEM((half,), jnp.int32),
            pltpu.VMEM((half,), jnp.int32),
            pltpu.VMEM((half,), jnp.float32),
            pltpu.VMEM((half,), jnp.float32),
            pltpu.VMEM((half,), jnp.float32),
            pltpu.VMEM((CR, dc), jnp.float32),
            pltpu.VMEM((CR, dc), jnp.float32),
            pltpu.VMEM((n_int, dc), jnp.float32),
            pltpu.SemaphoreType.DMA,
            pltpu.SemaphoreType.DMA,
        ],
    )
    def sc_scatter(i1h, i2h, i3h, e1h, e2h, e3h, items, out,
                   i1v, i2v, i3v, e1v, e2v, e3v, ib0, ib1, acc, sm0, sm1):
        c = lax.axis_index("c")
        s = lax.axis_index("s")
        w = s * 2 + c
        rg = w // NCG
        cg = w % NCG
        rbase = rg * rows_rg

        zero = jnp.zeros((LANES,), jnp.float32)

        def zero_body(t, carry):
            for r in range(nreg):
                acc[t, pl.ds(r * LANES, LANES)] = zero
            return carry

        lax.fori_loop(0, n_int, zero_body, 0)

        dn = lax.GatherDimensionNumbers(
            offset_dims=(), collapsed_slice_dims=(0,),
            start_index_map=(0,))
        pib = lax.GatherScatterMode.PROMISE_IN_BOUNDS
        lane = lax.iota(jnp.int32, LANES)

        def fire(cstart, buf, sem):
            roff = jnp.minimum(cstart, n - CR)
            pltpu.async_copy(
                items.at[pl.ds(nbase + roff, CR), pl.ds(cg * dc, dc)],
                buf, sem)

        def wait(buf, sem):
            pltpu.make_async_copy(
                items.at[pl.ds(0, CR), pl.ds(cg * dc, dc)], buf, sem).wait()

        def compute(cstart, sb, buf):
            roff = jnp.minimum(cstart, n - CR)
            q = roff - sb

            def group_body(g, carry2):
                iv1 = i1v[pl.ds(q + g * LANES, LANES)]
                iv2 = i2v[pl.ds(q + g * LANES, LANES)]
                iv3 = i3v[pl.ds(q + g * LANES, LANES)]
                ev1 = e1v[pl.ds(q + g * LANES, LANES)]
                ev2 = e2v[pl.ds(q + g * LANES, LANES)]
                ev3 = e3v[pl.ds(q + g * LANES, LANES)]

                def item_body(j, carry3):
                    p = jnp.full((LANES, 1), j, jnp.int32)
                    ok = (roff + g * LANES + j) >= cstart
                    t1 = lax.gather(iv1, p, dn, (1,), mode=pib)
                    t2 = lax.gather(iv2, p, dn, (1,), mode=pib)
                    t3 = lax.gather(iv3, p, dn, (1,), mode=pib)
                    g1 = jnp.where(ok, lax.gather(ev1, p, dn, (1,), mode=pib), 0.0)
                    g2 = jnp.where(ok, lax.gather(ev2, p, dn, (1,), mode=pib), 0.0)
                    g3 = jnp.where(ok, lax.gather(ev3, p, dn, (1,), mode=pib), 0.0)
                    jj = g * LANES + j
                    for r in range(nreg):
                        cols = lane + r * LANES
                        rv = buf[jj, pl.ds(r * LANES, LANES)]
                        plsc.addupdate_scatter(acc, [t1, cols], rv * g1)
                        plsc.addupdate_scatter(acc, [t2, cols], rv * g2)
                        plsc.addupdate_scatter(acc, [t3, cols], rv * g3)
                    return carry3

                return lax.fori_loop(0, LANES, item_body, carry2)

            lax.fori_loop(0, CR // LANES, group_body, 0)

        for h in range(2):
            hbase = rbase + h * half
            sb = jnp.minimum(hbase, n - half)
            pltpu.sync_copy(i1h.at[pl.ds(sb, half)], i1v)
            pltpu.sync_copy(i2h.at[pl.ds(sb, half)], i2v)
            pltpu.sync_copy(i3h.at[pl.ds(sb, half)], i3v)
            pltpu.sync_copy(e1h.at[pl.ds(sb, half)], e1v)
            pltpu.sync_copy(e2h.at[pl.ds(sb, half)], e2v)
            pltpu.sync_copy(e3h.at[pl.ds(sb, half)], e3v)

            fire(hbase, ib0, sm0)

            def pair_body(m, carry, hbase=hbase, sb=sb):
                c0 = hbase + (2 * m) * CR
                wait(ib0, sm0)
                fire(c0 + CR, ib1, sm1)
                compute(c0, sb, ib0)
                wait(ib1, sm1)

                @pl.when(m + 1 < nch // 2)
                def _prefetch():
                    fire(c0 + 2 * CR, ib0, sm0)

                compute(c0 + CR, sb, ib1)
                return carry

            lax.fori_loop(0, nch // 2, pair_body, 0)

        pltpu.sync_copy(acc, out.at[rg, cg])

    return sc_scatter


def _pass2_kernel(item_ref, idx_ref, acca_ref, accb_ref, ssa_ref, ssb_ref,
                  wb_ref, out_ref):
    blk = item_ref[...]
    B, d = blk.shape
    n_int = acca_ref.shape[2]

    red = jnp.sum(acca_ref[...], axis=0) + jnp.sum(accb_ref[...], axis=0)
    ncg = red.shape[0]
    full = jnp.concatenate([red[i] for i in range(ncg)], axis=1)
    ss = ssa_ref[...] + ssb_ref[...]
    denom = jnp.where(ss == 0.0, 1.0, ss)
    intent_new = full / denom[:, None]

    iota = lax.broadcasted_iota(jnp.int32, (B, n_int), 1)
    oh1 = iota == idx_ref[0, 0, :][:, None]
    oh2 = iota == idx_ref[0, 1, :][:, None]
    oh3 = iota == idx_ref[0, 2, :][:, None]

    a2 = lax.dot_general(blk * wb_ref[...][None, :], intent_new,
                         (((1,), (1,)), ((), ())),
                         preferred_element_type=jnp.float32)
    t1 = _leaky(jnp.sum(jnp.where(oh1, a2, 0.0), axis=1))
    t2 = _leaky(jnp.sum(jnp.where(oh2, a2, 0.0), axis=1))
    t3 = _leaky(jnp.sum(jnp.where(oh3, a2, 0.0), axis=1))
    f1 = jnp.exp(t1)
    f2 = jnp.exp(t2)
    f3 = jnp.exp(t3)
    srow = f1 + f2 + f3
    w1 = f1 / srow
    w2 = f2 / srow
    w3 = f3 / srow

    P2 = (w1[:, None] * oh1.astype(jnp.float32)
          + w2[:, None] * oh2.astype(jnp.float32)
          + w3[:, None] * oh3.astype(jnp.float32))
    nei = lax.dot_general(P2, intent_new, (((1,), (0,)), ((), ())),
                          preferred_element_type=jnp.float32)
    out_ref[...] = ALPHA * blk + (1.0 - ALPHA) * nei


def _pick_block(n):
    for b in range(2048, 0, -8):
        if n % b == 0:
            return b
    return n


def kernel(item_emb, n_items, intent_emb, n_intents, Wq, Wk, wa, wb):
    n, d = item_emb.shape
    n_int = intent_emb.shape[0]
    B = _pick_block(n)
    nb = n // B

    k_mat = pl.pallas_call(
        _prep_kernel,
        out_shape=jax.ShapeDtypeStruct((n_int, d), jnp.float32),
    )(intent_emb, Wk)

    def run_pass1(nblk, boff):
        return pl.pallas_call(
            _pass1_kernel,
            grid=(nblk,),
            in_specs=[
                pl.BlockSpec((B, d), lambda i: (i + boff, 0)),
                pl.BlockSpec((d, d), lambda i: (0, 0)),
                pl.BlockSpec((n_int, d), lambda i: (0, 0)),
                pl.BlockSpec((n_int, d), lambda i: (0, 0)),
                pl.BlockSpec((d,), lambda i: (0,)),
            ],
            out_specs=[
                pl.BlockSpec((1, 3, B), lambda i: (i, 0, 0)),
                pl.BlockSpec((1, 3, B), lambda i: (i, 0, 0)),
                pl.BlockSpec((n_int,), lambda i: (0,)),
            ],
            out_shape=[
                jax.ShapeDtypeStruct((nblk, 3, B), jnp.int32),
                jax.ShapeDtypeStruct((nblk, 3, B), jnp.float32),
                jax.ShapeDtypeStruct((n_int,), jnp.float32),
            ],
            compiler_params=pltpu.CompilerParams(
                dimension_semantics=("arbitrary",)),
        )(item_emb, Wq, k_mat, intent_emb, wa)

    def run_sc(idx_h, ev_h, nsub, nbase):
        idxf = jnp.transpose(idx_h, (1, 0, 2)).reshape(3, nsub)
        evf = jnp.transpose(ev_h, (1, 0, 2)).reshape(3, nsub)
        return _make_sc_scatter(nsub, d, n_int, nbase)(
            idxf[0], idxf[1], idxf[2], evf[0], evf[1], evf[2], item_emb)

    nba = nb // 2
    nbb = nb - nba
    idxa, eva, ssa = run_pass1(nba, 0)
    idxb, evb, ssb = run_pass1(nbb, nba)
    acca = run_sc(idxa, eva, nba * B, 0)
    accb = run_sc(idxb, evb, nbb * B, nba * B)

    idx = jnp.concatenate([idxa, idxb], axis=0)

    out = pl.pallas_call(
        _pass2_kernel,
        grid=(nb,),
        in_specs=[
            pl.BlockSpec((B, d), lambda i: (i, 0)),
            pl.BlockSpec((1, 3, B), lambda i: (i, 0, 0)),
            pl.BlockSpec((NRG, d // 128, n_int, 128),
                         lambda i: (0, 0, 0, 0)),
            pl.BlockSpec((NRG, d // 128, n_int, 128),
                         lambda i: (0, 0, 0, 0)),
            pl.BlockSpec((n_int,), lambda i: (0,)),
            pl.BlockSpec((n_int,), lambda i: (0,)),
            pl.BlockSpec((d,), lambda i: (0,)),
        ],
        out_specs=pl.BlockSpec((B, d), lambda i: (i, 0)),
        out_shape=jax.ShapeDtypeStruct((n, d), jnp.float32),
        compiler_params=pltpu.CompilerParams(
            dimension_semantics=("arbitrary",)),
    )(item_emb, idx, acca, accb, ssa, ssb, wb)
    return out


# speedup vs baseline: 1.1964x; 1.0198x over previous
import functools

import jax
import jax.numpy as jnp
from jax import lax
from jax.experimental import pallas as pl
from jax.experimental.pallas import tpu as pltpu
from jax.experimental.pallas import tpu_sc as plsc

ALPHA = 0.5
NEG = -3.0e38
LANES = 16
NW = 32


def _leaky(x):
    return jnp.where(x > 0, x, 0.2 * x)


def _prep_kernel(intent_ref, wk_ref, k_ref):
    k_ref[...] = lax.dot_general(intent_ref[...], wk_ref[...],
                                 (((1,), (1,)), ((), ())),
                                 preferred_element_type=jnp.float32)


def _pass1_kernel(item_ref, wq_ref, k_ref, intent_ref, wa_ref,
                  idx_ref, ev_ref, ss_ref):
    i = pl.program_id(0)
    blk = item_ref[...]
    B, d = blk.shape
    n_int = k_ref.shape[0]

    q = lax.dot_general(blk, wq_ref[...], (((1,), (1,)), ((), ())),
                        preferred_element_type=jnp.float32)
    logits = lax.dot_general(q, k_ref[...], (((1,), (1,)), ((), ())),
                             preferred_element_type=jnp.float32)
    iota = lax.broadcasted_iota(jnp.int32, (B, n_int), 1)

    def amax(l):
        m = jnp.max(l, axis=1, keepdims=True)
        return jnp.min(jnp.where(l == m, iota, n_int + 1), axis=1)

    i1 = amax(logits)
    l2 = jnp.where(iota == i1[:, None], NEG, logits)
    i2 = amax(l2)
    l3 = jnp.where(iota == i2[:, None], NEG, l2)
    i3 = amax(l3)

    smin = jnp.minimum(jnp.minimum(i1, i2), i3)
    smax = jnp.maximum(jnp.maximum(i1, i2), i3)
    smid = i1 + i2 + i3 - smin - smax

    oh1 = iota == smin[:, None]
    oh2 = iota == smid[:, None]
    oh3 = iota == smax[:, None]

    idx_ref[0, 0, :] = smin
    idx_ref[0, 1, :] = smid
    idx_ref[0, 2, :] = smax

    a1 = lax.dot_general(blk * wa_ref[...][None, :], intent_ref[...],
                         (((1,), (1,)), ((), ())),
                         preferred_element_type=jnp.float32)
    e1 = jnp.exp(_leaky(jnp.sum(jnp.where(oh1, a1, 0.0), axis=1)))
    e2 = jnp.exp(_leaky(jnp.sum(jnp.where(oh2, a1, 0.0), axis=1)))
    e3 = jnp.exp(_leaky(jnp.sum(jnp.where(oh3, a1, 0.0), axis=1)))

    ev_ref[0, 0, :] = e1
    ev_ref[0, 1, :] = e2
    ev_ref[0, 2, :] = e3

    @pl.when(i == 0)
    def _init():
        ss_ref[...] = jnp.zeros_like(ss_ref)

    ss_ref[...] += (jnp.sum(jnp.where(oh1, e1[:, None], 0.0), axis=0)
                    + jnp.sum(jnp.where(oh2, e2[:, None], 0.0), axis=0)
                    + jnp.sum(jnp.where(oh3, e3[:, None], 0.0), axis=0))


CR = 128
NRG = 8


def _make_sc_scatter(n, d, n_int, nbase):
    NCG = d // 128
    dc = 128
    nreg = dc // LANES
    rpg = (n + NRG - 1) // NRG
    rows_rg = (rpg + 4 * CR - 1) // (4 * CR) * (4 * CR)
    half = rows_rg // 2
    nch = half // CR
    mesh = plsc.VectorSubcoreMesh(core_axis_name="c", subcore_axis_name="s")

    @functools.partial(
        pl.kernel, mesh=mesh,
        compiler_params=pltpu.CompilerParams(needs_layout_passes=False),
        out_type=jax.ShapeDtypeStruct((NRG, NCG, n_int, dc), jnp.float32),
        scratch_types=[
            pltpu.VMEM((half,), jnp.int32),
            pltpu.VMEM((half,), jnp.int32),
            pltpu.VMEM((half,), jnp.int32),
            pltpu.VMEM((half,), jnp.float32),
            pltpu.VMEM((half,), jnp.float32),
            pltpu.VMEM((half,), jnp.float32),
            pltpu.VMEM((CR, dc), jnp.float32),
            pltpu.VMEM((CR, dc), jnp.float32),
            pltpu.VMEM((n_int, dc), jnp.float32),
            pltpu.SemaphoreType.DMA,
            pltpu.SemaphoreType.DMA,
        ],
    )
    def sc_scatter(i1h, i2h, i3h, e1h, e2h, e3h, items, out,
                   i1v, i2v, i3v, e1v, e2v, e3v, ib0, ib1, acc, sm0, sm1):
        c = lax.axis_index("c")
        s = lax.axis_index("s")
        w = s * 2 + c
        rg = w // NCG
        cg = w % NCG
        rbase = rg * rows_rg

        zero = jnp.zeros((LANES,), jnp.float32)

        def zero_body(t, carry):
            for r in range(nreg):
                acc[t, pl.ds(r * LANES, LANES)] = zero
            return carry

        lax.fori_loop(0, n_int, zero_body, 0)

        dn = lax.GatherDimensionNumbers(
            offset_dims=(), collapsed_slice_dims=(0,),
            start_index_map=(0,))
        pib = lax.GatherScatterMode.PROMISE_IN_BOUNDS
        lane = lax.iota(jnp.int32, LANES)

        def fire(cstart, buf, sem):
            roff = jnp.minimum(cstart, n - CR)
            pltpu.async_copy(
                items.at[pl.ds(nbase + roff, CR), pl.ds(cg * dc, dc)],
                buf, sem)

        def wait(buf, sem):
            pltpu.make_async_copy(
                items.at[pl.ds(0, CR), pl.ds(cg * dc, dc)], buf, sem).wait()

        def compute(cstart, sb, buf):
            roff = jnp.minimum(cstart, n - CR)
            q = roff - sb

            def group_body(g, carry2):
                iv1 = i1v[pl.ds(q + g * LANES, LANES)]
                iv2 = i2v[pl.ds(q + g * LANES, LANES)]
                iv3 = i3v[pl.ds(q + g * LANES, LANES)]
                ev1 = e1v[pl.ds(q + g * LANES, LANES)]
                ev2 = e2v[pl.ds(q + g * LANES, LANES)]
                ev3 = e3v[pl.ds(q + g * LANES, LANES)]

                def item_body(j, carry3):
                    p = jnp.full((LANES, 1), j, jnp.int32)
                    ok = (roff + g * LANES + j) >= cstart
                    t1 = lax.gather(iv1, p, dn, (1,), mode=pib)
                    t2 = lax.gather(iv2, p, dn, (1,), mode=pib)
                    t3 = lax.gather(iv3, p, dn, (1,), mode=pib)
                    g1 = jnp.where(ok, lax.gather(ev1, p, dn, (1,), mode=pib), 0.0)
                    g2 = jnp.where(ok, lax.gather(ev2, p, dn, (1,), mode=pib), 0.0)
                    g3 = jnp.where(ok, lax.gather(ev3, p, dn, (1,), mode=pib), 0.0)
                    jj = g * LANES + j
                    for r in range(nreg):
                        cols = lane + r * LANES
                        rv = buf[jj, pl.ds(r * LANES, LANES)]
                        plsc.addupdate_scatter(acc, [t1, cols], rv * g1)
                        plsc.addupdate_scatter(acc, [t2, cols], rv * g2)
                        plsc.addupdate_scatter(acc, [t3, cols], rv * g3)
                    return carry3

                return lax.fori_loop(0, LANES, item_body, carry2)

            lax.fori_loop(0, CR // LANES, group_body, 0)

        for h in range(2):
            hbase = rbase + h * half
            sb = jnp.minimum(hbase, n - half)
            pltpu.sync_copy(i1h.at[pl.ds(sb, half)], i1v)
            pltpu.sync_copy(i2h.at[pl.ds(sb, half)], i2v)
            pltpu.sync_copy(i3h.at[pl.ds(sb, half)], i3v)
            pltpu.sync_copy(e1h.at[pl.ds(sb, half)], e1v)
            pltpu.sync_copy(e2h.at[pl.ds(sb, half)], e2v)
            pltpu.sync_copy(e3h.at[pl.ds(sb, half)], e3v)

            fire(hbase, ib0, sm0)

            def pair_body(m, carry, hbase=hbase, sb=sb):
                c0 = hbase + (2 * m) * CR
                wait(ib0, sm0)
                fire(c0 + CR, ib1, sm1)
                compute(c0, sb, ib0)
                wait(ib1, sm1)

                @pl.when(m + 1 < nch // 2)
                def _prefetch():
                    fire(c0 + 2 * CR, ib0, sm0)

                compute(c0 + CR, sb, ib1)
                return carry

            lax.fori_loop(0, nch // 2, pair_body, 0)

        pltpu.sync_copy(acc, out.at[rg, cg])

    return sc_scatter


def _pass2_kernel(item_ref, idx_ref, *rest):
    nsplit = (len(rest) - 2) // 2
    acc_refs = rest[:nsplit]
    ss_refs = rest[nsplit:2 * nsplit]
    wb_ref = rest[2 * nsplit]
    out_ref = rest[2 * nsplit + 1]
    blk = item_ref[...]
    B, d = blk.shape
    n_int = acc_refs[0].shape[2]

    red = jnp.sum(acc_refs[0][...], axis=0)
    for a in acc_refs[1:]:
        red = red + jnp.sum(a[...], axis=0)
    ncg = red.shape[0]
    full = jnp.concatenate([red[i] for i in range(ncg)], axis=1)
    ss = ss_refs[0][...]
    for sref in ss_refs[1:]:
        ss = ss + sref[...]
    denom = jnp.where(ss == 0.0, 1.0, ss)
    intent_new = full / denom[:, None]

    iota = lax.broadcasted_iota(jnp.int32, (B, n_int), 1)
    oh1 = iota == idx_ref[0, 0, :][:, None]
    oh2 = iota == idx_ref[0, 1, :][:, None]
    oh3 = iota == idx_ref[0, 2, :][:, None]

    a2 = lax.dot_general(blk * wb_ref[...][None, :], intent_new,
                         (((1,), (1,)), ((), ())),
                         preferred_element_type=jnp.float32)
    t1 = _leaky(jnp.sum(jnp.where(oh1, a2, 0.0), axis=1))
    t2 = _leaky(jnp.sum(jnp.where(oh2, a2, 0.0), axis=1))
    t3 = _leaky(jnp.sum(jnp.where(oh3, a2, 0.0), axis=1))
    f1 = jnp.exp(t1)
    f2 = jnp.exp(t2)
    f3 = jnp.exp(t3)
    srow = f1 + f2 + f3
    w1 = f1 / srow
    w2 = f2 / srow
    w3 = f3 / srow

    P2 = (w1[:, None] * oh1.astype(jnp.float32)
          + w2[:, None] * oh2.astype(jnp.float32)
          + w3[:, None] * oh3.astype(jnp.float32))
    nei = lax.dot_general(P2, intent_new, (((1,), (0,)), ((), ())),
                          preferred_element_type=jnp.float32)
    out_ref[...] = ALPHA * blk + (1.0 - ALPHA) * nei


def _pick_block(n):
    for b in range(2048, 0, -8):
        if n % b == 0:
            return b
    return n


def kernel(item_emb, n_items, intent_emb, n_intents, Wq, Wk, wa, wb):
    n, d = item_emb.shape
    n_int = intent_emb.shape[0]
    B = _pick_block(n)
    nb = n // B

    k_mat = pl.pallas_call(
        _prep_kernel,
        out_shape=jax.ShapeDtypeStruct((n_int, d), jnp.float32),
    )(intent_emb, Wk)

    def run_pass1(nblk, boff):
        return pl.pallas_call(
            _pass1_kernel,
            grid=(nblk,),
            in_specs=[
                pl.BlockSpec((B, d), lambda i: (i + boff, 0)),
                pl.BlockSpec((d, d), lambda i: (0, 0)),
                pl.BlockSpec((n_int, d), lambda i: (0, 0)),
                pl.BlockSpec((n_int, d), lambda i: (0, 0)),
                pl.BlockSpec((d,), lambda i: (0,)),
            ],
            out_specs=[
                pl.BlockSpec((1, 3, B), lambda i: (i, 0, 0)),
                pl.BlockSpec((1, 3, B), lambda i: (i, 0, 0)),
                pl.BlockSpec((n_int,), lambda i: (0,)),
            ],
            out_shape=[
                jax.ShapeDtypeStruct((nblk, 3, B), jnp.int32),
                jax.ShapeDtypeStruct((nblk, 3, B), jnp.float32),
                jax.ShapeDtypeStruct((n_int,), jnp.float32),
            ],
            compiler_params=pltpu.CompilerParams(
                dimension_semantics=("arbitrary",)),
        )(item_emb, Wq, k_mat, intent_emb, wa)

    def run_sc(idx_h, ev_h, nsub, nbase):
        idxf = jnp.transpose(idx_h, (1, 0, 2)).reshape(3, nsub)
        evf = jnp.transpose(ev_h, (1, 0, 2)).reshape(3, nsub)
        return _make_sc_scatter(nsub, d, n_int, nbase)(
            idxf[0], idxf[1], idxf[2], evf[0], evf[1], evf[2], item_emb)

    nsplit = 4 if nb >= 8 else 1
    base_nb, rem = nb // nsplit, nb % nsplit
    counts = [base_nb + (1 if i < rem else 0) for i in range(nsplit)]
    offs = [sum(counts[:i]) for i in range(nsplit)]

    p1 = [run_pass1(nblk, boff) for nblk, boff in zip(counts, offs)]
    accs = [run_sc(idx_h, ev_h, nblk * B, boff * B)
            for (idx_h, ev_h, _), nblk, boff in zip(p1, counts, offs)]
    sss = [r[2] for r in p1]

    idx = jnp.concatenate([r[0] for r in p1], axis=0)

    acc_spec = pl.BlockSpec((NRG, d // 128, n_int, 128),
                            lambda i: (0, 0, 0, 0))
    ss_spec = pl.BlockSpec((n_int,), lambda i: (0,))
    out = pl.pallas_call(
        _pass2_kernel,
        grid=(nb,),
        in_specs=[
            pl.BlockSpec((B, d), lambda i: (i, 0)),
            pl.BlockSpec((1, 3, B), lambda i: (i, 0, 0)),
        ] + [acc_spec] * nsplit + [ss_spec] * nsplit + [
            pl.BlockSpec((d,), lambda i: (0,)),
        ],
        out_specs=pl.BlockSpec((B, d), lambda i: (i, 0)),
        out_shape=jax.ShapeDtypeStruct((n, d), jnp.float32),
        compiler_params=pltpu.CompilerParams(
            dimension_semantics=("arbitrary",)),
    )(item_emb, idx, *accs, *sss, wb)
    return out


# speedup vs baseline: 1.2127x; 1.0136x over previous
import functools

import jax
import jax.numpy as jnp
from jax import lax
from jax.experimental import pallas as pl
from jax.experimental.pallas import tpu as pltpu
from jax.experimental.pallas import tpu_sc as plsc

ALPHA = 0.5
NEG = -3.0e38
LANES = 16
NW = 32


def _leaky(x):
    return jnp.where(x > 0, x, 0.2 * x)


def _prep_kernel(intent_ref, wk_ref, k_ref):
    k_ref[...] = lax.dot_general(intent_ref[...], wk_ref[...],
                                 (((1,), (1,)), ((), ())),
                                 preferred_element_type=jnp.float32)


def _pass1_kernel(item_ref, wq_ref, k_ref, intent_ref, wa_ref,
                  idx_ref, ev_ref, ss_ref):
    i = pl.program_id(0)
    blk = item_ref[...]
    B, d = blk.shape
    n_int = k_ref.shape[0]

    q = lax.dot_general(blk, wq_ref[...], (((1,), (1,)), ((), ())),
                        preferred_element_type=jnp.float32)
    logits = lax.dot_general(q, k_ref[...], (((1,), (1,)), ((), ())),
                             preferred_element_type=jnp.float32)
    iota = lax.broadcasted_iota(jnp.int32, (B, n_int), 1)

    def amax(l):
        m = jnp.max(l, axis=1, keepdims=True)
        return jnp.min(jnp.where(l == m, iota, n_int + 1), axis=1)

    i1 = amax(logits)
    l2 = jnp.where(iota == i1[:, None], NEG, logits)
    i2 = amax(l2)
    l3 = jnp.where(iota == i2[:, None], NEG, l2)
    i3 = amax(l3)

    smin = jnp.minimum(jnp.minimum(i1, i2), i3)
    smax = jnp.maximum(jnp.maximum(i1, i2), i3)
    smid = i1 + i2 + i3 - smin - smax

    oh1 = iota == smin[:, None]
    oh2 = iota == smid[:, None]
    oh3 = iota == smax[:, None]

    idx_ref[0, 0, :] = smin
    idx_ref[0, 1, :] = smid
    idx_ref[0, 2, :] = smax

    a1 = lax.dot_general(blk * wa_ref[...][None, :], intent_ref[...],
                         (((1,), (1,)), ((), ())),
                         preferred_element_type=jnp.float32)
    e1 = jnp.exp(_leaky(jnp.sum(jnp.where(oh1, a1, 0.0), axis=1)))
    e2 = jnp.exp(_leaky(jnp.sum(jnp.where(oh2, a1, 0.0), axis=1)))
    e3 = jnp.exp(_leaky(jnp.sum(jnp.where(oh3, a1, 0.0), axis=1)))

    ev_ref[0, 0, :] = e1
    ev_ref[0, 1, :] = e2
    ev_ref[0, 2, :] = e3

    @pl.when(i == 0)
    def _init():
        ss_ref[...] = jnp.zeros_like(ss_ref)

    ss_ref[...] += (jnp.sum(jnp.where(oh1, e1[:, None], 0.0), axis=0)
                    + jnp.sum(jnp.where(oh2, e2[:, None], 0.0), axis=0)
                    + jnp.sum(jnp.where(oh3, e3[:, None], 0.0), axis=0))


CR = 128
NRG = 8


def _make_sc_scatter(n, d, n_int, nbase):
    NCG = d // 128
    dc = 128
    nreg = dc // LANES
    rpg = (n + NRG - 1) // NRG
    rows_rg = (rpg + 4 * CR - 1) // (4 * CR) * (4 * CR)
    half = rows_rg // 2
    nch = half // CR
    mesh = plsc.VectorSubcoreMesh(core_axis_name="c", subcore_axis_name="s")

    @functools.partial(
        pl.kernel, mesh=mesh,
        compiler_params=pltpu.CompilerParams(needs_layout_passes=False),
        out_type=jax.ShapeDtypeStruct((NRG, NCG, n_int, dc), jnp.float32),
        scratch_types=[
            pltpu.VMEM((half,), jnp.int32),
            pltpu.VMEM((half,), jnp.int32),
            pltpu.VMEM((half,), jnp.int32),
            pltpu.VMEM((half,), jnp.float32),
            pltpu.VMEM((half,), jnp.float32),
            pltpu.VMEM((half,), jnp.float32),
            pltpu.VMEM((CR, dc), jnp.float32),
            pltpu.VMEM((CR, dc), jnp.float32),
            pltpu.VMEM((n_int, dc), jnp.float32),
            pltpu.SemaphoreType.DMA,
            pltpu.SemaphoreType.DMA,
        ],
    )
    def sc_scatter(i1h, i2h, i3h, e1h, e2h, e3h, items, out,
                   i1v, i2v, i3v, e1v, e2v, e3v, ib0, ib1, acc, sm0, sm1):
        c = lax.axis_index("c")
        s = lax.axis_index("s")
        w = s * 2 + c
        rg = w // NCG
        cg = w % NCG
        rbase = rg * rows_rg

        zero = jnp.zeros((LANES,), jnp.float32)

        def zero_body(t, carry):
            for r in range(nreg):
                acc[t, pl.ds(r * LANES, LANES)] = zero
            return carry

        lax.fori_loop(0, n_int, zero_body, 0)

        dn = lax.GatherDimensionNumbers(
            offset_dims=(), collapsed_slice_dims=(0,),
            start_index_map=(0,))
        pib = lax.GatherScatterMode.PROMISE_IN_BOUNDS
        lane = lax.iota(jnp.int32, LANES)

        def fire(cstart, buf, sem):
            roff = jnp.minimum(cstart, n - CR)
            pltpu.async_copy(
                items.at[pl.ds(nbase + roff, CR), pl.ds(cg * dc, dc)],
                buf, sem)

        def wait(buf, sem):
            pltpu.make_async_copy(
                items.at[pl.ds(0, CR), pl.ds(cg * dc, dc)], buf, sem).wait()

        def compute(cstart, sb, buf):
            roff = jnp.minimum(cstart, n - CR)
            q = roff - sb

            def group_body(g, carry2):
                iv1 = i1v[pl.ds(q + g * LANES, LANES)]
                iv2 = i2v[pl.ds(q + g * LANES, LANES)]
                iv3 = i3v[pl.ds(q + g * LANES, LANES)]
                ev1 = e1v[pl.ds(q + g * LANES, LANES)]
                ev2 = e2v[pl.ds(q + g * LANES, LANES)]
                ev3 = e3v[pl.ds(q + g * LANES, LANES)]

                def item_body(j, carry3):
                    p = jnp.full((LANES, 1), j, jnp.int32)
                    ok = (roff + g * LANES + j) >= cstart
                    t1 = lax.gather(iv1, p, dn, (1,), mode=pib)
                    t2 = lax.gather(iv2, p, dn, (1,), mode=pib)
                    t3 = lax.gather(iv3, p, dn, (1,), mode=pib)
                    g1 = jnp.where(ok, lax.gather(ev1, p, dn, (1,), mode=pib), 0.0)
                    g2 = jnp.where(ok, lax.gather(ev2, p, dn, (1,), mode=pib), 0.0)
                    g3 = jnp.where(ok, lax.gather(ev3, p, dn, (1,), mode=pib), 0.0)
                    jj = g * LANES + j
                    for r in range(nreg):
                        cols = lane + r * LANES
                        rv = buf[jj, pl.ds(r * LANES, LANES)]
                        plsc.addupdate_scatter(acc, [t1, cols], rv * g1)
                        plsc.addupdate_scatter(acc, [t2, cols], rv * g2)
                        plsc.addupdate_scatter(acc, [t3, cols], rv * g3)
                    return carry3

                return lax.fori_loop(0, LANES, item_body, carry2)

            lax.fori_loop(0, CR // LANES, group_body, 0)

        for h in range(2):
            hbase = rbase + h * half
            sb = jnp.minimum(hbase, n - half)
            pltpu.sync_copy(i1h.at[pl.ds(sb, half)], i1v)
            pltpu.sync_copy(i2h.at[pl.ds(sb, half)], i2v)
            pltpu.sync_copy(i3h.at[pl.ds(sb, half)], i3v)
            pltpu.sync_copy(e1h.at[pl.ds(sb, half)], e1v)
            pltpu.sync_copy(e2h.at[pl.ds(sb, half)], e2v)
            pltpu.sync_copy(e3h.at[pl.ds(sb, half)], e3v)

            fire(hbase, ib0, sm0)

            def pair_body(m, carry, hbase=hbase, sb=sb):
                c0 = hbase + (2 * m) * CR
                wait(ib0, sm0)
                fire(c0 + CR, ib1, sm1)
                compute(c0, sb, ib0)
                wait(ib1, sm1)

                @pl.when(m + 1 < nch // 2)
                def _prefetch():
                    fire(c0 + 2 * CR, ib0, sm0)

                compute(c0 + CR, sb, ib1)
                return carry

            lax.fori_loop(0, nch // 2, pair_body, 0)

        pltpu.sync_copy(acc, out.at[rg, cg])

    return sc_scatter


def _pass2_kernel(item_ref, idx_ref, *rest):
    nsplit = (len(rest) - 2) // 2
    acc_refs = rest[:nsplit]
    ss_refs = rest[nsplit:2 * nsplit]
    wb_ref = rest[2 * nsplit]
    out_ref = rest[2 * nsplit + 1]
    blk = item_ref[...]
    B, d = blk.shape
    n_int = acc_refs[0].shape[2]

    red = jnp.sum(acc_refs[0][...], axis=0)
    for a in acc_refs[1:]:
        red = red + jnp.sum(a[...], axis=0)
    ncg = red.shape[0]
    full = jnp.concatenate([red[i] for i in range(ncg)], axis=1)
    ss = ss_refs[0][...]
    for sref in ss_refs[1:]:
        ss = ss + sref[...]
    denom = jnp.where(ss == 0.0, 1.0, ss)
    intent_new = full / denom[:, None]

    iota = lax.broadcasted_iota(jnp.int32, (B, n_int), 1)
    oh1 = iota == idx_ref[0, 0, :][:, None]
    oh2 = iota == idx_ref[0, 1, :][:, None]
    oh3 = iota == idx_ref[0, 2, :][:, None]

    a2 = lax.dot_general(blk * wb_ref[...][None, :], intent_new,
                         (((1,), (1,)), ((), ())),
                         preferred_element_type=jnp.float32)
    t1 = _leaky(jnp.sum(jnp.where(oh1, a2, 0.0), axis=1))
    t2 = _leaky(jnp.sum(jnp.where(oh2, a2, 0.0), axis=1))
    t3 = _leaky(jnp.sum(jnp.where(oh3, a2, 0.0), axis=1))
    f1 = jnp.exp(t1)
    f2 = jnp.exp(t2)
    f3 = jnp.exp(t3)
    srow = f1 + f2 + f3
    w1 = f1 / srow
    w2 = f2 / srow
    w3 = f3 / srow

    P2 = (w1[:, None] * oh1.astype(jnp.float32)
          + w2[:, None] * oh2.astype(jnp.float32)
          + w3[:, None] * oh3.astype(jnp.float32))
    nei = lax.dot_general(P2, intent_new, (((1,), (0,)), ((), ())),
                          preferred_element_type=jnp.float32)
    out_ref[...] = ALPHA * blk + (1.0 - ALPHA) * nei


def _pick_block(n):
    for b in range(2048, 0, -8):
        if n % b == 0:
            return b
    return n


def kernel(item_emb, n_items, intent_emb, n_intents, Wq, Wk, wa, wb):
    n, d = item_emb.shape
    n_int = intent_emb.shape[0]
    B = _pick_block(n)
    nb = n // B

    k_mat = pl.pallas_call(
        _prep_kernel,
        out_shape=jax.ShapeDtypeStruct((n_int, d), jnp.float32),
    )(intent_emb, Wk)

    def run_pass1(nblk, boff):
        return pl.pallas_call(
            _pass1_kernel,
            grid=(nblk,),
            in_specs=[
                pl.BlockSpec((B, d), lambda i: (i + boff, 0)),
                pl.BlockSpec((d, d), lambda i: (0, 0)),
                pl.BlockSpec((n_int, d), lambda i: (0, 0)),
                pl.BlockSpec((n_int, d), lambda i: (0, 0)),
                pl.BlockSpec((d,), lambda i: (0,)),
            ],
            out_specs=[
                pl.BlockSpec((1, 3, B), lambda i: (i, 0, 0)),
                pl.BlockSpec((1, 3, B), lambda i: (i, 0, 0)),
                pl.BlockSpec((n_int,), lambda i: (0,)),
            ],
            out_shape=[
                jax.ShapeDtypeStruct((nblk, 3, B), jnp.int32),
                jax.ShapeDtypeStruct((nblk, 3, B), jnp.float32),
                jax.ShapeDtypeStruct((n_int,), jnp.float32),
            ],
            compiler_params=pltpu.CompilerParams(
                dimension_semantics=("arbitrary",)),
        )(item_emb, Wq, k_mat, intent_emb, wa)

    def run_sc(idx_h, ev_h, nsub, nbase):
        idxf = jnp.transpose(idx_h, (1, 0, 2)).reshape(3, nsub)
        evf = jnp.transpose(ev_h, (1, 0, 2)).reshape(3, nsub)
        return _make_sc_scatter(nsub, d, n_int, nbase)(
            idxf[0], idxf[1], idxf[2], evf[0], evf[1], evf[2], item_emb)

    nsplit = 3 if nb >= 6 else 1
    base_nb, rem = nb // nsplit, nb % nsplit
    counts = [base_nb + (1 if i < rem else 0) for i in range(nsplit)]
    offs = [sum(counts[:i]) for i in range(nsplit)]

    p1 = [run_pass1(nblk, boff) for nblk, boff in zip(counts, offs)]
    accs = [run_sc(idx_h, ev_h, nblk * B, boff * B)
            for (idx_h, ev_h, _), nblk, boff in zip(p1, counts, offs)]
    sss = [r[2] for r in p1]

    idx = jnp.concatenate([r[0] for r in p1], axis=0)

    acc_spec = pl.BlockSpec((NRG, d // 128, n_int, 128),
                            lambda i: (0, 0, 0, 0))
    ss_spec = pl.BlockSpec((n_int,), lambda i: (0,))
    out = pl.pallas_call(
        _pass2_kernel,
        grid=(nb,),
        in_specs=[
            pl.BlockSpec((B, d), lambda i: (i, 0)),
            pl.BlockSpec((1, 3, B), lambda i: (i, 0, 0)),
        ] + [acc_spec] * nsplit + [ss_spec] * nsplit + [
            pl.BlockSpec((d,), lambda i: (0,)),
        ],
        out_specs=pl.BlockSpec((B, d), lambda i: (i, 0)),
        out_shape=jax.ShapeDtypeStruct((n, d), jnp.float32),
        compiler_params=pltpu.CompilerParams(
            dimension_semantics=("arbitrary",)),
    )(item_emb, idx, *accs, *sss, wb)
    return out
